# baseline (speedup 1.0000x reference)
import functools

import jax
import jax.numpy as jnp
from jax import lax
from jax.experimental import pallas as pl
from jax.experimental.pallas import tpu as pltpu
from jax.experimental.pallas import tpu_sc as plsc

N_NODES = 100000
N_EDGES = 6400000

NC = 2
NS = 16
NW = NC * NS
CH = 128
EPT = 200704
E_PAD = EPT * NW
N_PAD = 100352
SLICE = N_PAD // NS
N_SPARE = N_PAD - N_NODES
E_ROWS = E_PAD // CH

KI = 16
NBLK = EPT // (KI * CH)

_f32 = jnp.float32


def _sc_deg_body(dst_hbm, degp_hbm, acc_ref, didx_v):
    c = lax.axis_index("c")
    s = lax.axis_index("s")
    wid = c * NS + s
    z16 = jnp.zeros((16,), _f32)
    ones16 = jnp.ones((16,), _f32)

    def zero_blk(i, _):
        acc_ref[pl.ds(i * 16, 16)] = z16
        return ()

    lax.fori_loop(0, N_PAD // 16, zero_blk, (), unroll=8)

    base_row = wid * (EPT // CH)

    def blk(g, _):
        pltpu.sync_copy(dst_hbm.at[pl.ds(base_row + g * KI, KI)], didx_v)

        def row(j, _):
            for k in range(8):
                di = didx_v[j, pl.ds(16 * k, 16)]
                plsc.addupdate_scatter(acc_ref, [di], ones16)
            return ()

        lax.fori_loop(0, KI, row, (), unroll=False)
        return ()

    lax.fori_loop(0, NBLK, blk, (), unroll=False)
    pltpu.sync_copy(acc_ref, degp_hbm.at[wid])


def _sc_edge_body(src_hbm, dst_hbm, tab_hbm, accp_hbm,
                  tab_sh, acc_ref, sidx_v, didx_v, val_v, stage_v, gsem):
    c = lax.axis_index("c")
    s = lax.axis_index("s")
    wid = c * NS + s
    sl = pl.ds(s * SLICE, SLICE)
    pltpu.sync_copy(tab_hbm.at[sl], stage_v)
    pltpu.sync_copy(stage_v, tab_sh.at[sl])

    z16 = jnp.zeros((16,), _f32)

    def zero_blk(i, _):
        acc_ref[pl.ds(i * 16, 16)] = z16
        return ()

    lax.fori_loop(0, N_PAD // 16, zero_blk, (), unroll=8)
    plsc.subcore_barrier()

    base_row = wid * (EPT // CH)

    def blk(g, _):
        row0 = base_row + g * KI
        pltpu.sync_copy(src_hbm.at[pl.ds(row0, KI)], sidx_v)
        pltpu.sync_copy(dst_hbm.at[pl.ds(row0, KI)], didx_v)
        gds = [
            pltpu.async_copy(tab_sh.at[sidx_v.at[j]], val_v.at[j], gsem)
            for j in range(KI)
        ]
        for cp in gds:
            cp.wait()

        def row(j, _):
            for k in range(8):
                di = didx_v[j, pl.ds(16 * k, 16)]
                vv = val_v[j, pl.ds(16 * k, 16)]
                plsc.addupdate_scatter(acc_ref, [di], vv)
            return ()

        lax.fori_loop(0, KI, row, (), unroll=False)
        return ()

    lax.fori_loop(0, NBLK, blk, (), unroll=False)
    pltpu.sync_copy(acc_ref, accp_hbm.at[wid])


@functools.lru_cache(maxsize=None)
def _sc_kernels():
    mesh = plsc.VectorSubcoreMesh(core_axis_name="c", subcore_axis_name="s",
                                  num_cores=NC, num_subcores=NS)
    params = pltpu.CompilerParams(needs_layout_passes=False)
    sc_deg = pl.kernel(
        _sc_deg_body,
        out_type=jax.ShapeDtypeStruct((NW, N_PAD), _f32),
        mesh=mesh,
        compiler_params=params,
        scratch_types=[
            pltpu.VMEM((N_PAD,), _f32),
            pltpu.VMEM((KI, CH), jnp.int32),
        ],
    )
    sc_edge = pl.kernel(
        _sc_edge_body,
        out_type=jax.ShapeDtypeStruct((NW, N_PAD), _f32),
        mesh=mesh,
        compiler_params=params,
        scratch_types=[
            pltpu.VMEM_SHARED((N_PAD,), _f32),
            pltpu.VMEM((N_PAD,), _f32),
            pltpu.VMEM((KI, CH), jnp.int32),
            pltpu.VMEM((KI, CH), jnp.int32),
            pltpu.VMEM((KI, CH), _f32),
            pltpu.VMEM((SLICE,), _f32),
            pltpu.SemaphoreType.DMA,
        ],
    )
    return sc_deg, sc_edge


def _stage_a_body(degp_ref, x_ref, dinv_ref, xs_ref):
    d = jnp.sum(degp_ref[...], axis=0) + 1.0
    dinv = lax.rsqrt(d)
    dinv_ref[...] = dinv
    xs_ref[...] = x_ref[...] * dinv


_stage_a = pl.pallas_call(
    _stage_a_body,
    out_shape=[jax.ShapeDtypeStruct((N_PAD // 128, 128), _f32)] * 2,
)


def _stage_b_body(a1p_ref, xs_ref, dinv_ref, w1_ref, b1_ref, w2_ref,
                  z0_ref, z1_ref):
    dinv = dinv_ref[...]
    y = dinv * (jnp.sum(a1p_ref[...], axis=0) + xs_ref[...])
    z0 = jnp.zeros_like(y)
    z1 = jnp.zeros_like(y)
    for k in range(16):
        h = jnp.maximum(y * w1_ref[0, k] + b1_ref[0, k], 0.0)
        z0 += h * w2_ref[k, 0]
        z1 += h * w2_ref[k, 1]
    z0_ref[...] = z0 * dinv
    z1_ref[...] = z1 * dinv


_stage_b = pl.pallas_call(
    _stage_b_body,
    in_specs=[
        pl.BlockSpec(),
        pl.BlockSpec(),
        pl.BlockSpec(),
        pl.BlockSpec(memory_space=pltpu.SMEM),
        pl.BlockSpec(memory_space=pltpu.SMEM),
        pl.BlockSpec(memory_space=pltpu.SMEM),
    ],
    out_shape=[jax.ShapeDtypeStruct((N_PAD // 128, 128), _f32)] * 2,
)


def _stage_c_body(a20_ref, a21_ref, zs0_ref, zs1_ref, dinv_ref, b2_ref,
                  o0_ref, o1_ref):
    dinv = dinv_ref[...]
    o0 = dinv * (jnp.sum(a20_ref[...], axis=0) + zs0_ref[...]) + b2_ref[0, 0]
    o1 = dinv * (jnp.sum(a21_ref[...], axis=0) + zs1_ref[...]) + b2_ref[0, 1]
    m = jnp.maximum(o0, o1)
    lse = m + jnp.log(jnp.exp(o0 - m) + jnp.exp(o1 - m))
    o0_ref[...] = o0 - lse
    o1_ref[...] = o1 - lse


_stage_c = pl.pallas_call(
    _stage_c_body,
    in_specs=[
        pl.BlockSpec(),
        pl.BlockSpec(),
        pl.BlockSpec(),
        pl.BlockSpec(),
        pl.BlockSpec(),
        pl.BlockSpec(memory_space=pltpu.SMEM),
    ],
    out_shape=[jax.ShapeDtypeStruct((N_PAD // 128, 128), _f32)] * 2,
)


def kernel(x, edge_index, W1, b1, W2, b2):
    sc_deg, sc_edge = _sc_kernels()
    nrows = N_PAD // 128
    src = edge_index[0].astype(jnp.int32)
    dst = edge_index[1].astype(jnp.int32)
    npad_e = E_PAD - N_EDGES
    pad = N_NODES + (jnp.arange(npad_e, dtype=jnp.int32) % N_SPARE)
    src2d = jnp.concatenate([src, pad]).reshape(E_ROWS, CH)
    dst2d = jnp.concatenate([dst, pad]).reshape(E_ROWS, CH)

    xpad = jnp.zeros((N_PAD,), _f32).at[:N_NODES].set(x[:, 0])

    degp = sc_deg(dst2d)
    dinv, xs = _stage_a(degp.reshape(NW, nrows, 128),
                        xpad.reshape(nrows, 128))

    acc1p = sc_edge(src2d, dst2d, xs.reshape(N_PAD))
    z0, z1 = _stage_b(acc1p.reshape(NW, nrows, 128), xs, dinv,
                      W1, b1.reshape(1, 16), W2)

    acc2p0 = sc_edge(src2d, dst2d, z0.reshape(N_PAD))
    acc2p1 = sc_edge(src2d, dst2d, z1.reshape(N_PAD))

    o0, o1 = _stage_c(acc2p0.reshape(NW, nrows, 128),
                      acc2p1.reshape(NW, nrows, 128),
                      z0, z1, dinv, b2.reshape(1, 2))

    return jnp.stack(
        [o0.reshape(N_PAD)[:N_NODES], o1.reshape(N_PAD)[:N_NODES]], axis=1)


# speedup vs baseline: 1.0041x; 1.0041x over previous
import functools

import jax
import jax.numpy as jnp
from jax import lax
from jax.experimental import pallas as pl
from jax.experimental.pallas import tpu as pltpu
from jax.experimental.pallas import tpu_sc as plsc

N_NODES = 100000
N_EDGES = 6400000

NC = 2
NS = 16
NW = NC * NS
CH = 128
EPT = 200704
E_PAD = EPT * NW
N_PAD = 100352
SLICE = N_PAD // NS
N_SPARE = N_PAD - N_NODES
E_ROWS = E_PAD // CH

KI = 16
NBLK = EPT // (KI * CH)

_f32 = jnp.float32


def _sc_deg_body(dst_hbm, degp_hbm, acc_ref, didx_f):
    c = lax.axis_index("c")
    s = lax.axis_index("s")
    wid = c * NS + s
    z16 = jnp.zeros((16,), _f32)
    ones16 = jnp.ones((16,), _f32)

    def zero_blk(i, _):
        acc_ref[pl.ds(i * 16, 16)] = z16
        return ()

    lax.fori_loop(0, N_PAD // 16, zero_blk, (), unroll=8)

    base = wid * EPT
    blk_e = KI * CH

    def blk(g, _):
        pltpu.sync_copy(dst_hbm.at[pl.ds(base + g * blk_e, blk_e)], didx_f)

        def chunk(i, _):
            di = didx_f[pl.ds(16 * i, 16)]
            plsc.addupdate_scatter(acc_ref, [di], ones16)
            return ()

        lax.fori_loop(0, blk_e // 16, chunk, (), unroll=16)
        return ()

    lax.fori_loop(0, NBLK, blk, (), unroll=False)
    pltpu.sync_copy(acc_ref, degp_hbm.at[wid])


def _sc_edge_body(src_hbm, dst_hbm, tab_hbm, accp_hbm,
                  tab_sh, acc_ref, sidx_v, didx_f, val_f, stage_v, gsem):
    c = lax.axis_index("c")
    s = lax.axis_index("s")
    wid = c * NS + s
    sl = pl.ds(s * SLICE, SLICE)
    pltpu.sync_copy(tab_hbm.at[sl], stage_v)
    pltpu.sync_copy(stage_v, tab_sh.at[sl])

    z16 = jnp.zeros((16,), _f32)

    def zero_blk(i, _):
        acc_ref[pl.ds(i * 16, 16)] = z16
        return ()

    lax.fori_loop(0, N_PAD // 16, zero_blk, (), unroll=8)
    plsc.subcore_barrier()

    base_row = wid * (EPT // CH)
    base = wid * EPT
    blk_e = KI * CH

    def blk(g, _):
        row0 = base_row + g * KI
        pltpu.sync_copy(src_hbm.at[pl.ds(row0, KI)], sidx_v)
        pltpu.sync_copy(dst_hbm.at[pl.ds(base + g * blk_e, blk_e)], didx_f)
        gds = [
            pltpu.async_copy(tab_sh.at[sidx_v.at[j]],
                             val_f.at[pl.ds(j * CH, CH)], gsem)
            for j in range(KI)
        ]
        for cp in gds:
            cp.wait()

        def chunk(i, _):
            di = didx_f[pl.ds(16 * i, 16)]
            vv = val_f[pl.ds(16 * i, 16)]
            plsc.addupdate_scatter(acc_ref, [di], vv)
            return ()

        lax.fori_loop(0, blk_e // 16, chunk, (), unroll=16)
        return ()

    lax.fori_loop(0, NBLK, blk, (), unroll=False)
    pltpu.sync_copy(acc_ref, accp_hbm.at[wid])


@functools.lru_cache(maxsize=None)
def _sc_kernels():
    mesh = plsc.VectorSubcoreMesh(core_axis_name="c", subcore_axis_name="s",
                                  num_cores=NC, num_subcores=NS)
    params = pltpu.CompilerParams(needs_layout_passes=False)
    sc_deg = pl.kernel(
        _sc_deg_body,
        out_type=jax.ShapeDtypeStruct((NW, N_PAD), _f32),
        mesh=mesh,
        compiler_params=params,
        scratch_types=[
            pltpu.VMEM((N_PAD,), _f32),
            pltpu.VMEM((KI * CH,), jnp.int32),
        ],
    )
    sc_edge = pl.kernel(
        _sc_edge_body,
        out_type=jax.ShapeDtypeStruct((NW, N_PAD), _f32),
        mesh=mesh,
        compiler_params=params,
        scratch_types=[
            pltpu.VMEM_SHARED((N_PAD,), _f32),
            pltpu.VMEM((N_PAD,), _f32),
            pltpu.VMEM((KI, CH), jnp.int32),
            pltpu.VMEM((KI * CH,), jnp.int32),
            pltpu.VMEM((KI * CH,), _f32),
            pltpu.VMEM((SLICE,), _f32),
            pltpu.SemaphoreType.DMA,
        ],
    )
    return sc_deg, sc_edge


def _stage_a_body(degp_ref, x_ref, dinv_ref, xs_ref):
    d = jnp.sum(degp_ref[...], axis=0) + 1.0
    dinv = lax.rsqrt(d)
    dinv_ref[...] = dinv
    xs_ref[...] = x_ref[...] * dinv


_stage_a = pl.pallas_call(
    _stage_a_body,
    out_shape=[jax.ShapeDtypeStruct((N_PAD // 128, 128), _f32)] * 2,
)


def _stage_b_body(a1p_ref, xs_ref, dinv_ref, w1_ref, b1_ref, w2_ref,
                  z0_ref, z1_ref):
    dinv = dinv_ref[...]
    y = dinv * (jnp.sum(a1p_ref[...], axis=0) + xs_ref[...])
    z0 = jnp.zeros_like(y)
    z1 = jnp.zeros_like(y)
    for k in range(16):
        h = jnp.maximum(y * w1_ref[0, k] + b1_ref[0, k], 0.0)
        z0 += h * w2_ref[k, 0]
        z1 += h * w2_ref[k, 1]
    z0_ref[...] = z0 * dinv
    z1_ref[...] = z1 * dinv


_stage_b = pl.pallas_call(
    _stage_b_body,
    in_specs=[
        pl.BlockSpec(),
        pl.BlockSpec(),
        pl.BlockSpec(),
        pl.BlockSpec(memory_space=pltpu.SMEM),
        pl.BlockSpec(memory_space=pltpu.SMEM),
        pl.BlockSpec(memory_space=pltpu.SMEM),
    ],
    out_shape=[jax.ShapeDtypeStruct((N_PAD // 128, 128), _f32)] * 2,
)


def _stage_c_body(a20_ref, a21_ref, zs0_ref, zs1_ref, dinv_ref, b2_ref,
                  o0_ref, o1_ref):
    dinv = dinv_ref[...]
    o0 = dinv * (jnp.sum(a20_ref[...], axis=0) + zs0_ref[...]) + b2_ref[0, 0]
    o1 = dinv * (jnp.sum(a21_ref[...], axis=0) + zs1_ref[...]) + b2_ref[0, 1]
    m = jnp.maximum(o0, o1)
    lse = m + jnp.log(jnp.exp(o0 - m) + jnp.exp(o1 - m))
    o0_ref[...] = o0 - lse
    o1_ref[...] = o1 - lse


_stage_c = pl.pallas_call(
    _stage_c_body,
    in_specs=[
        pl.BlockSpec(),
        pl.BlockSpec(),
        pl.BlockSpec(),
        pl.BlockSpec(),
        pl.BlockSpec(),
        pl.BlockSpec(memory_space=pltpu.SMEM),
    ],
    out_shape=[jax.ShapeDtypeStruct((N_PAD // 128, 128), _f32)] * 2,
)


def kernel(x, edge_index, W1, b1, W2, b2):
    sc_deg, sc_edge = _sc_kernels()
    nrows = N_PAD // 128
    src = edge_index[0].astype(jnp.int32)
    dst = edge_index[1].astype(jnp.int32)
    npad_e = E_PAD - N_EDGES
    pad = N_NODES + (jnp.arange(npad_e, dtype=jnp.int32) % N_SPARE)
    src2d = jnp.concatenate([src, pad]).reshape(E_ROWS, CH)
    dst1d = jnp.concatenate([dst, pad])

    xpad = jnp.zeros((N_PAD,), _f32).at[:N_NODES].set(x[:, 0])

    degp = sc_deg(dst1d)
    dinv, xs = _stage_a(degp.reshape(NW, nrows, 128),
                        xpad.reshape(nrows, 128))

    acc1p = sc_edge(src2d, dst1d, xs.reshape(N_PAD))
    z0, z1 = _stage_b(acc1p.reshape(NW, nrows, 128), xs, dinv,
                      W1, b1.reshape(1, 16), W2)

    acc2p0 = sc_edge(src2d, dst1d, z0.reshape(N_PAD))
    acc2p1 = sc_edge(src2d, dst1d, z1.reshape(N_PAD))

    o0, o1 = _stage_c(acc2p0.reshape(NW, nrows, 128),
                      acc2p1.reshape(NW, nrows, 128),
                      z0, z1, dinv, b2.reshape(1, 2))

    return jnp.stack(
        [o0.reshape(N_PAD)[:N_NODES], o1.reshape(N_PAD)[:N_NODES]], axis=1)


# speedup vs baseline: 1.2077x; 1.2028x over previous
import functools

import jax
import jax.numpy as jnp
from jax import lax
from jax.experimental import pallas as pl
from jax.experimental.pallas import tpu as pltpu
from jax.experimental.pallas import tpu_sc as plsc

N_NODES = 100000
N_EDGES = 6400000

NC = 2
NS = 16
NW = NC * NS
CH = 128
EPT = 200704
E_PAD = EPT * NW
N_PAD = 100352
SLICE = N_PAD // NS
N_SPARE = N_PAD - N_NODES
E_ROWS = E_PAD // CH

KI = 32
NBLK = EPT // (KI * CH)

_f32 = jnp.float32


def _sc_deg_body(dst_hbm, degp_hbm, acc_ref, didx_f):
    c = lax.axis_index("c")
    s = lax.axis_index("s")
    wid = c * NS + s
    z16 = jnp.zeros((16,), _f32)
    ones16 = jnp.ones((16,), _f32)

    def zero_blk(i, _):
        acc_ref[pl.ds(i * 16, 16)] = z16
        return ()

    lax.fori_loop(0, N_PAD // 16, zero_blk, (), unroll=8)

    base = wid * EPT
    blk_e = KI * CH

    def blk(g, _):
        pltpu.sync_copy(dst_hbm.at[pl.ds(base + g * blk_e, blk_e)], didx_f)

        def chunk(i, _):
            di = didx_f[pl.ds(16 * i, 16)]
            plsc.addupdate_scatter(acc_ref, [di], ones16)
            return ()

        lax.fori_loop(0, blk_e // 16, chunk, (), unroll=16)
        return ()

    lax.fori_loop(0, NBLK, blk, (), unroll=False)
    pltpu.sync_copy(acc_ref, degp_hbm.at[wid])


def _sc_edge_body(src_hbm, dst_hbm, tab_hbm, accp_hbm,
                  tab_sh, acc_ref, sidx_v, didx_f, val_f, stage_v, gsem):
    c = lax.axis_index("c")
    s = lax.axis_index("s")
    wid = c * NS + s
    sl = pl.ds(s * SLICE, SLICE)
    pltpu.sync_copy(tab_hbm.at[sl], stage_v)
    pltpu.sync_copy(stage_v, tab_sh.at[sl])

    z16 = jnp.zeros((16,), _f32)

    def zero_blk(i, _):
        acc_ref[pl.ds(i * 16, 16)] = z16
        return ()

    lax.fori_loop(0, N_PAD // 16, zero_blk, (), unroll=8)
    plsc.subcore_barrier()

    base_row = wid * (EPT // CH)
    base = wid * EPT
    blk_e = KI * CH

    def blk(g, _):
        row0 = base_row + g * KI
        pltpu.sync_copy(src_hbm.at[pl.ds(row0, KI)], sidx_v)
        pltpu.sync_copy(dst_hbm.at[pl.ds(base + g * blk_e, blk_e)], didx_f)
        gds = [
            pltpu.async_copy(tab_sh.at[sidx_v.at[j]],
                             val_f.at[pl.ds(j * CH, CH)], gsem)
            for j in range(KI)
        ]
        for cp in gds:
            cp.wait()

        def chunk(i, _):
            di = didx_f[pl.ds(16 * i, 16)]
            vv = val_f[pl.ds(16 * i, 16)]
            plsc.addupdate_scatter(acc_ref, [di], vv)
            return ()

        lax.fori_loop(0, blk_e // 16, chunk, (), unroll=16)
        return ()

    lax.fori_loop(0, NBLK, blk, (), unroll=False)
    pltpu.sync_copy(acc_ref, accp_hbm.at[wid])


@functools.lru_cache(maxsize=None)
def _sc_kernels():
    mesh = plsc.VectorSubcoreMesh(core_axis_name="c", subcore_axis_name="s",
                                  num_cores=NC, num_subcores=NS)
    params = pltpu.CompilerParams(needs_layout_passes=False)
    sc_deg = pl.kernel(
        _sc_deg_body,
        out_type=jax.ShapeDtypeStruct((NW, N_PAD), _f32),
        mesh=mesh,
        compiler_params=params,
        scratch_types=[
            pltpu.VMEM((N_PAD,), _f32),
            pltpu.VMEM((KI * CH,), jnp.int32),
        ],
    )
    sc_edge = pl.kernel(
        _sc_edge_body,
        out_type=jax.ShapeDtypeStruct((NW, N_PAD), _f32),
        mesh=mesh,
        compiler_params=params,
        scratch_types=[
            pltpu.VMEM_SHARED((N_PAD,), _f32),
            pltpu.VMEM((N_PAD,), _f32),
            pltpu.VMEM((KI, CH), jnp.int32),
            pltpu.VMEM((KI * CH,), jnp.int32),
            pltpu.VMEM((KI * CH,), _f32),
            pltpu.VMEM((SLICE,), _f32),
            pltpu.SemaphoreType.DMA,
        ],
    )
    return sc_deg, sc_edge


def _stage_a_body(degp_ref, x_ref, dinv_ref, xs_ref):
    d = jnp.sum(degp_ref[...], axis=0) + 1.0
    dinv = lax.rsqrt(d)
    dinv_ref[...] = dinv
    xs_ref[...] = x_ref[...] * dinv


_stage_a = pl.pallas_call(
    _stage_a_body,
    out_shape=[jax.ShapeDtypeStruct((N_PAD // 128, 128), _f32)] * 2,
)


def _stage_b_body(a1p_ref, xs_ref, dinv_ref, w1_ref, b1_ref, w2_ref,
                  z0_ref, z1_ref):
    dinv = dinv_ref[...]
    y = dinv * (jnp.sum(a1p_ref[...], axis=0) + xs_ref[...])
    z0 = jnp.zeros_like(y)
    z1 = jnp.zeros_like(y)
    for k in range(16):
        h = jnp.maximum(y * w1_ref[0, k] + b1_ref[0, k], 0.0)
        z0 += h * w2_ref[k, 0]
        z1 += h * w2_ref[k, 1]
    z0_ref[...] = z0 * dinv
    z1_ref[...] = z1 * dinv


_stage_b = pl.pallas_call(
    _stage_b_body,
    in_specs=[
        pl.BlockSpec(),
        pl.BlockSpec(),
        pl.BlockSpec(),
        pl.BlockSpec(memory_space=pltpu.SMEM),
        pl.BlockSpec(memory_space=pltpu.SMEM),
        pl.BlockSpec(memory_space=pltpu.SMEM),
    ],
    out_shape=[jax.ShapeDtypeStruct((N_PAD // 128, 128), _f32)] * 2,
)


def _stage_c_body(a20_ref, a21_ref, zs0_ref, zs1_ref, dinv_ref, b2_ref,
                  o0_ref, o1_ref):
    dinv = dinv_ref[...]
    o0 = dinv * (jnp.sum(a20_ref[...], axis=0) + zs0_ref[...]) + b2_ref[0, 0]
    o1 = dinv * (jnp.sum(a21_ref[...], axis=0) + zs1_ref[...]) + b2_ref[0, 1]
    m = jnp.maximum(o0, o1)
    lse = m + jnp.log(jnp.exp(o0 - m) + jnp.exp(o1 - m))
    o0_ref[...] = o0 - lse
    o1_ref[...] = o1 - lse


_stage_c = pl.pallas_call(
    _stage_c_body,
    in_specs=[
        pl.BlockSpec(),
        pl.BlockSpec(),
        pl.BlockSpec(),
        pl.BlockSpec(),
        pl.BlockSpec(),
        pl.BlockSpec(memory_space=pltpu.SMEM),
    ],
    out_shape=[jax.ShapeDtypeStruct((N_PAD // 128, 128), _f32)] * 2,
)


def kernel(x, edge_index, W1, b1, W2, b2):
    sc_deg, sc_edge = _sc_kernels()
    nrows = N_PAD // 128
    src = edge_index[0].astype(jnp.int32)
    dst = edge_index[1].astype(jnp.int32)
    npad_e = E_PAD - N_EDGES
    pad = N_NODES + (jnp.arange(npad_e, dtype=jnp.int32) % N_SPARE)
    src2d = jnp.concatenate([src, pad]).reshape(E_ROWS, CH)
    dst1d = jnp.concatenate([dst, pad])

    xpad = jnp.zeros((N_PAD,), _f32).at[:N_NODES].set(x[:, 0])

    degp = sc_deg(dst1d)
    dinv, xs = _stage_a(degp.reshape(NW, nrows, 128),
                        xpad.reshape(nrows, 128))

    acc1p = sc_edge(src2d, dst1d, xs.reshape(N_PAD))
    z0, z1 = _stage_b(acc1p.reshape(NW, nrows, 128), xs, dinv,
                      W1, b1.reshape(1, 16), W2)

    acc2p0 = sc_edge(src2d, dst1d, z0.reshape(N_PAD))
    acc2p1 = sc_edge(src2d, dst1d, z1.reshape(N_PAD))

    o0, o1 = _stage_c(acc2p0.reshape(NW, nrows, 128),
                      acc2p1.reshape(NW, nrows, 128),
                      z0, z1, dinv, b2.reshape(1, 2))

    return jnp.stack(
        [o0.reshape(N_PAD)[:N_NODES], o1.reshape(N_PAD)[:N_NODES]], axis=1)


# speedup vs baseline: 1.3233x; 1.0957x over previous
import functools

import jax
import jax.numpy as jnp
from jax import lax
from jax.experimental import pallas as pl
from jax.experimental.pallas import tpu as pltpu
from jax.experimental.pallas import tpu_sc as plsc

N_NODES = 100000
N_EDGES = 6400000

NC = 2
NS = 16
NW = NC * NS
CH = 128
EPT = 200704
E_PAD = EPT * NW
N_PAD = 100352
SLICE = N_PAD // NS
N_SPARE = N_PAD - N_NODES
E_ROWS = E_PAD // CH

KI = 56
NBLK = EPT // (KI * CH)

_f32 = jnp.float32


def _sc_deg_body(dst_hbm, degp_hbm, acc_ref, didx_f):
    c = lax.axis_index("c")
    s = lax.axis_index("s")
    wid = c * NS + s
    z16 = jnp.zeros((16,), _f32)
    ones16 = jnp.ones((16,), _f32)

    def zero_blk(i, _):
        acc_ref[pl.ds(i * 16, 16)] = z16
        return ()

    lax.fori_loop(0, N_PAD // 16, zero_blk, (), unroll=8)

    base = wid * EPT
    blk_e = KI * CH

    def blk(g, _):
        pltpu.sync_copy(dst_hbm.at[pl.ds(base + g * blk_e, blk_e)], didx_f)

        def chunk(i, _):
            di = didx_f[pl.ds(16 * i, 16)]
            plsc.addupdate_scatter(acc_ref, [di], ones16)
            return ()

        lax.fori_loop(0, blk_e // 16, chunk, (), unroll=16)
        return ()

    lax.fori_loop(0, NBLK, blk, (), unroll=False)
    pltpu.sync_copy(acc_ref, degp_hbm.at[wid])


def _sc_edge_body(src_hbm, dst_hbm, tab_hbm, accp_hbm,
                  tab_sh, acc_ref, sidx_v, didx_f, val_f, stage_v, gsem):
    c = lax.axis_index("c")
    s = lax.axis_index("s")
    wid = c * NS + s
    quart = SLICE // 4
    for p in range(4):
        slp = pl.ds(s * SLICE + p * quart, quart)
        pltpu.sync_copy(tab_hbm.at[slp], stage_v)
        pltpu.sync_copy(stage_v, tab_sh.at[slp])

    z16 = jnp.zeros((16,), _f32)

    def zero_blk(i, _):
        acc_ref[pl.ds(i * 16, 16)] = z16
        return ()

    lax.fori_loop(0, N_PAD // 16, zero_blk, (), unroll=8)
    plsc.subcore_barrier()

    base_row = wid * (EPT // CH)
    base = wid * EPT
    blk_e = KI * CH

    def blk(g, _):
        row0 = base_row + g * KI
        pltpu.sync_copy(src_hbm.at[pl.ds(row0, KI)], sidx_v)
        pltpu.sync_copy(dst_hbm.at[pl.ds(base + g * blk_e, blk_e)], didx_f)
        gds = [
            pltpu.async_copy(tab_sh.at[sidx_v.at[j]],
                             val_f.at[pl.ds(j * CH, CH)], gsem)
            for j in range(KI)
        ]
        for cp in gds:
            cp.wait()

        def chunk(i, _):
            di = didx_f[pl.ds(16 * i, 16)]
            vv = val_f[pl.ds(16 * i, 16)]
            plsc.addupdate_scatter(acc_ref, [di], vv)
            return ()

        lax.fori_loop(0, blk_e // 16, chunk, (), unroll=16)
        return ()

    lax.fori_loop(0, NBLK, blk, (), unroll=False)
    pltpu.sync_copy(acc_ref, accp_hbm.at[wid])


@functools.lru_cache(maxsize=None)
def _sc_kernels():
    mesh = plsc.VectorSubcoreMesh(core_axis_name="c", subcore_axis_name="s",
                                  num_cores=NC, num_subcores=NS)
    params = pltpu.CompilerParams(needs_layout_passes=False)
    sc_deg = pl.kernel(
        _sc_deg_body,
        out_type=jax.ShapeDtypeStruct((NW, N_PAD), _f32),
        mesh=mesh,
        compiler_params=params,
        scratch_types=[
            pltpu.VMEM((N_PAD,), _f32),
            pltpu.VMEM((KI * CH,), jnp.int32),
        ],
    )
    sc_edge = pl.kernel(
        _sc_edge_body,
        out_type=jax.ShapeDtypeStruct((NW, N_PAD), _f32),
        mesh=mesh,
        compiler_params=params,
        scratch_types=[
            pltpu.VMEM_SHARED((N_PAD,), _f32),
            pltpu.VMEM((N_PAD,), _f32),
            pltpu.VMEM((KI, CH), jnp.int32),
            pltpu.VMEM((KI * CH,), jnp.int32),
            pltpu.VMEM((KI * CH,), _f32),
            pltpu.VMEM((SLICE // 4,), _f32),
            pltpu.SemaphoreType.DMA,
        ],
    )
    return sc_deg, sc_edge


def _stage_a_body(degp_ref, x_ref, dinv_ref, xs_ref):
    d = jnp.sum(degp_ref[...], axis=0) + 1.0
    dinv = lax.rsqrt(d)
    dinv_ref[...] = dinv
    xs_ref[...] = x_ref[...] * dinv


_stage_a = pl.pallas_call(
    _stage_a_body,
    out_shape=[jax.ShapeDtypeStruct((N_PAD // 128, 128), _f32)] * 2,
)


def _stage_b_body(a1p_ref, xs_ref, dinv_ref, w1_ref, b1_ref, w2_ref,
                  z0_ref, z1_ref):
    dinv = dinv_ref[...]
    y = dinv * (jnp.sum(a1p_ref[...], axis=0) + xs_ref[...])
    z0 = jnp.zeros_like(y)
    z1 = jnp.zeros_like(y)
    for k in range(16):
        h = jnp.maximum(y * w1_ref[0, k] + b1_ref[0, k], 0.0)
        z0 += h * w2_ref[k, 0]
        z1 += h * w2_ref[k, 1]
    z0_ref[...] = z0 * dinv
    z1_ref[...] = z1 * dinv


_stage_b = pl.pallas_call(
    _stage_b_body,
    in_specs=[
        pl.BlockSpec(),
        pl.BlockSpec(),
        pl.BlockSpec(),
        pl.BlockSpec(memory_space=pltpu.SMEM),
        pl.BlockSpec(memory_space=pltpu.SMEM),
        pl.BlockSpec(memory_space=pltpu.SMEM),
    ],
    out_shape=[jax.ShapeDtypeStruct((N_PAD // 128, 128), _f32)] * 2,
)


def _stage_c_body(a20_ref, a21_ref, zs0_ref, zs1_ref, dinv_ref, b2_ref,
                  o0_ref, o1_ref):
    dinv = dinv_ref[...]
    o0 = dinv * (jnp.sum(a20_ref[...], axis=0) + zs0_ref[...]) + b2_ref[0, 0]
    o1 = dinv * (jnp.sum(a21_ref[...], axis=0) + zs1_ref[...]) + b2_ref[0, 1]
    m = jnp.maximum(o0, o1)
    lse = m + jnp.log(jnp.exp(o0 - m) + jnp.exp(o1 - m))
    o0_ref[...] = o0 - lse
    o1_ref[...] = o1 - lse


_stage_c = pl.pallas_call(
    _stage_c_body,
    in_specs=[
        pl.BlockSpec(),
        pl.BlockSpec(),
        pl.BlockSpec(),
        pl.BlockSpec(),
        pl.BlockSpec(),
        pl.BlockSpec(memory_space=pltpu.SMEM),
    ],
    out_shape=[jax.ShapeDtypeStruct((N_PAD // 128, 128), _f32)] * 2,
)


def kernel(x, edge_index, W1, b1, W2, b2):
    sc_deg, sc_edge = _sc_kernels()
    nrows = N_PAD // 128
    src = edge_index[0].astype(jnp.int32)
    dst = edge_index[1].astype(jnp.int32)
    npad_e = E_PAD - N_EDGES
    pad = N_NODES + (jnp.arange(npad_e, dtype=jnp.int32) % N_SPARE)
    src2d = jnp.concatenate([src, pad]).reshape(E_ROWS, CH)
    dst1d = jnp.concatenate([dst, pad])

    xpad = jnp.zeros((N_PAD,), _f32).at[:N_NODES].set(x[:, 0])

    degp = sc_deg(dst1d)
    dinv, xs = _stage_a(degp.reshape(NW, nrows, 128),
                        xpad.reshape(nrows, 128))

    acc1p = sc_edge(src2d, dst1d, xs.reshape(N_PAD))
    z0, z1 = _stage_b(acc1p.reshape(NW, nrows, 128), xs, dinv,
                      W1, b1.reshape(1, 16), W2)

    acc2p0 = sc_edge(src2d, dst1d, z0.reshape(N_PAD))
    acc2p1 = sc_edge(src2d, dst1d, z1.reshape(N_PAD))

    o0, o1 = _stage_c(acc2p0.reshape(NW, nrows, 128),
                      acc2p1.reshape(NW, nrows, 128),
                      z0, z1, dinv, b2.reshape(1, 2))

    return jnp.stack(
        [o0.reshape(N_PAD)[:N_NODES], o1.reshape(N_PAD)[:N_NODES]], axis=1)
